# Initial kernel scaffold; baseline (speedup 1.0000x reference)
import jax
import jax.numpy as jnp
from jax.experimental import pallas as pl


def kernel(x, w1, cb1, g1, b1, w2, cb2, g2, b2):
    raise NotImplementedError("write your pallas kernel here")


# speedup vs baseline: 5.3654x; 5.3654x over previous
import functools

import jax
import jax.numpy as jnp
from jax import lax
from jax.experimental import pallas as pl
from jax.experimental.pallas import tpu as pltpu

_EPS = 1e-5


def _conv9(z, w_ref, h, wp, c):
    m = h * wp
    acc = None
    for dy in range(3):
        for dx in range(3):
            s = dy * wp + dx
            t = dy * 3 + dx
            part = jnp.dot(z[s:s + m, :], w_ref[t * c:(t + 1) * c, :],
                           preferred_element_type=jnp.float32)
            acc = part if acc is None else acc + part
    return acc


def _col_mask(m, c, wp, w):
    r = lax.broadcasted_iota(jnp.int32, (m, c), 0)
    return (r % wp) < w


def _bn_coeffs(s_ref, ss_ref, g_ref, b_ref, inv_m):
    tot = jnp.sum(s_ref[...], axis=0)
    tot2 = jnp.sum(ss_ref[...], axis=0)
    mean = tot * inv_m
    var = jnp.maximum(tot2 * inv_m - mean * mean, 0.0)
    inv = lax.rsqrt(var + _EPS)
    scale = g_ref[...] * inv
    shift = b_ref[...] - mean * scale
    return scale, shift


def _stage1(xp_ref, w_ref, y_ref, s_ref, ss_ref, *, h, w, wp, c):
    m = h * wp
    rows_pad = y_ref.shape[1]
    acc = _conv9(xp_ref[0], w_ref, h, wp, c)
    accm = jnp.where(_col_mask(m, c, wp, w), acc, 0.0)
    lo = wp + 1
    y_ref[0, 0:lo, :] = jnp.zeros((lo, c), y_ref.dtype)
    y_ref[0, lo:lo + m, :] = accm.astype(y_ref.dtype)
    y_ref[0, lo + m:, :] = jnp.zeros((rows_pad - lo - m, c), y_ref.dtype)
    s_ref[0] = jnp.sum(accm, axis=0, keepdims=True)
    ss_ref[0] = jnp.sum(accm * accm, axis=0, keepdims=True)


def _stage2(y1_ref, w_ref, s_ref, ss_ref, g_ref, b_ref,
            y2_ref, s2_ref, ss2_ref, *, h, w, wp, c, inv_m):
    m = h * wp
    rows_pad = y1_ref.shape[1]
    scale, shift = _bn_coeffs(s_ref, ss_ref, g_ref, b_ref, inv_m)
    y1 = y1_ref[0]
    r = lax.broadcasted_iota(jnp.int32, (rows_pad, c), 0)
    lo = wp + 1
    interior = (r >= lo) & (r < lo + m) & (((r - lo) % wp) < w)
    z = jnp.where(interior, jnp.maximum(y1 * scale + shift, 0.0), 0.0)
    z = z.astype(jnp.bfloat16)
    acc = _conv9(z, w_ref, h, wp, c)
    accm = jnp.where(_col_mask(m, c, wp, w), acc, 0.0)
    y2_ref[0] = accm.astype(y2_ref.dtype)
    s2_ref[0] = jnp.sum(accm, axis=0, keepdims=True)
    ss2_ref[0] = jnp.sum(accm * accm, axis=0, keepdims=True)


def _stage3(y2_ref, s_ref, ss_ref, g_ref, b_ref, res_ref, o_ref, *, inv_m):
    scale, shift = _bn_coeffs(s_ref, ss_ref, g_ref, b_ref, inv_m)
    y = y2_ref[0] * scale + shift + res_ref[0]
    o_ref[0] = jnp.maximum(y, 0.0).astype(o_ref.dtype)


def kernel(x, w1, cb1, g1, b1, w2, cb2, g2, b2):
    del cb1, cb2
    n, c, h, w = x.shape
    wp = w + 2
    m = h * wp
    rows_img = (h + 2) * wp
    need = rows_img + 2
    rows_pad = -(-need // 8) * 8

    x_nhwc = jnp.transpose(x, (0, 2, 3, 1)).astype(jnp.float32)
    xp = jnp.pad(x_nhwc, ((0, 0), (1, 1), (1, 1), (0, 0)))
    xp = xp.reshape(n, rows_img, c)
    xp = jnp.pad(xp, ((0, 0), (0, rows_pad - rows_img), (0, 0)))
    xp = xp.astype(jnp.bfloat16)
    res = jnp.pad(x_nhwc, ((0, 0), (0, 0), (0, 2), (0, 0)))
    res = res.reshape(n, m, c)
    w1m = w1.reshape(9 * c, c).astype(jnp.bfloat16)
    w2m = w2.reshape(9 * c, c).astype(jnp.bfloat16)
    g1v = g1.astype(jnp.float32).reshape(1, c)
    b1v = b1.astype(jnp.float32).reshape(1, c)
    g2v = g2.astype(jnp.float32).reshape(1, c)
    b2v = b2.astype(jnp.float32).reshape(1, c)
    inv_m = 1.0 / (n * h * w)

    img_spec = pl.BlockSpec((1, rows_pad, c), lambda i: (i, 0, 0))
    row_spec = pl.BlockSpec((1, m, c), lambda i: (i, 0, 0))
    stat_spec = pl.BlockSpec((1, 1, c), lambda i: (i, 0, 0))
    stat_full = pl.BlockSpec((n, 1, c), lambda i: (0, 0, 0))
    w_spec = pl.BlockSpec((9 * c, c), lambda i: (0, 0))
    vec_spec = pl.BlockSpec((1, c), lambda i: (0, 0))
    params = pltpu.CompilerParams(dimension_semantics=("parallel",))
    f32 = jnp.float32

    y1, s1, ss1 = pl.pallas_call(
        functools.partial(_stage1, h=h, w=w, wp=wp, c=c),
        grid=(n,),
        in_specs=[img_spec, w_spec],
        out_specs=(img_spec, stat_spec, stat_spec),
        out_shape=(jax.ShapeDtypeStruct((n, rows_pad, c), f32),
                   jax.ShapeDtypeStruct((n, 1, c), f32),
                   jax.ShapeDtypeStruct((n, 1, c), f32)),
        compiler_params=params,
    )(xp, w1m)

    y2, s2, ss2 = pl.pallas_call(
        functools.partial(_stage2, h=h, w=w, wp=wp, c=c, inv_m=inv_m),
        grid=(n,),
        in_specs=[img_spec, w_spec, stat_full, stat_full, vec_spec, vec_spec],
        out_specs=(row_spec, stat_spec, stat_spec),
        out_shape=(jax.ShapeDtypeStruct((n, m, c), f32),
                   jax.ShapeDtypeStruct((n, 1, c), f32),
                   jax.ShapeDtypeStruct((n, 1, c), f32)),
        compiler_params=params,
    )(y1, w2m, s1, ss1, g1v, b1v)

    out = pl.pallas_call(
        functools.partial(_stage3, inv_m=inv_m),
        grid=(n,),
        in_specs=[row_spec, stat_full, stat_full, vec_spec, vec_spec, row_spec],
        out_specs=row_spec,
        out_shape=jax.ShapeDtypeStruct((n, m, c), f32),
        compiler_params=params,
    )(y2, s2, ss2, g2v, b2v, res)

    out = out.reshape(n, h, wp, c)[:, :, :w, :]
    return jnp.transpose(out, (0, 3, 1, 2))


# speedup vs baseline: 5.7031x; 1.0629x over previous
import functools

import jax
import jax.numpy as jnp
from jax import lax
from jax.experimental import pallas as pl
from jax.experimental.pallas import tpu as pltpu

_EPS = 1e-5


def _conv9(z, w_ref, h, wp, c):
    m = h * wp
    acc = None
    for dy in range(3):
        for dx in range(3):
            s = dy * wp + dx
            t = dy * 3 + dx
            part = jnp.dot(z[s:s + m, :], w_ref[t * c:(t + 1) * c, :],
                           preferred_element_type=jnp.float32)
            acc = part if acc is None else acc + part
    return acc


def _col_mask(m, c, wp, w):
    r = lax.broadcasted_iota(jnp.int32, (m, c), 0)
    return (r % wp) < w


def _bn_coeffs(s_ref, ss_ref, g_ref, b_ref, inv_m):
    tot = jnp.sum(s_ref[...], axis=0)
    tot2 = jnp.sum(ss_ref[...], axis=0)
    mean = tot * inv_m
    var = jnp.maximum(tot2 * inv_m - mean * mean, 0.0)
    inv = lax.rsqrt(var + _EPS)
    scale = g_ref[...] * inv
    shift = b_ref[...] - mean * scale
    return scale, shift


def _stage1(xp_ref, w_ref, y_ref, s_ref, ss_ref, *, h, w, wp, c):
    m = h * wp
    rows_pad = y_ref.shape[1]
    acc = _conv9(xp_ref[0], w_ref, h, wp, c)
    accm = jnp.where(_col_mask(m, c, wp, w), acc, 0.0)
    lo = wp + 1
    y_ref[0, 0:lo, :] = jnp.zeros((lo, c), y_ref.dtype)
    y_ref[0, lo:lo + m, :] = accm.astype(y_ref.dtype)
    y_ref[0, lo + m:, :] = jnp.zeros((rows_pad - lo - m, c), y_ref.dtype)
    s_ref[0] = jnp.sum(accm, axis=0, keepdims=True)
    ss_ref[0] = jnp.sum(accm * accm, axis=0, keepdims=True)


def _stage2(y1_ref, w_ref, s_ref, ss_ref, g_ref, b_ref,
            y2_ref, s2_ref, ss2_ref, *, h, w, wp, c, inv_m):
    m = h * wp
    rows_pad = y1_ref.shape[1]
    scale, shift = _bn_coeffs(s_ref, ss_ref, g_ref, b_ref, inv_m)
    y1 = y1_ref[0]
    r = lax.broadcasted_iota(jnp.int32, (rows_pad, c), 0)
    lo = wp + 1
    interior = (r >= lo) & (r < lo + m) & (((r - lo) % wp) < w)
    z = jnp.where(interior, jnp.maximum(y1 * scale + shift, 0.0), 0.0)
    z = z.astype(jnp.bfloat16)
    acc = _conv9(z, w_ref, h, wp, c)
    accm = jnp.where(_col_mask(m, c, wp, w), acc, 0.0)
    y2_ref[0] = accm.astype(y2_ref.dtype)
    s2_ref[0] = jnp.sum(accm, axis=0, keepdims=True)
    ss2_ref[0] = jnp.sum(accm * accm, axis=0, keepdims=True)


def _stage3(y2_ref, s_ref, ss_ref, g_ref, b_ref, res_ref, o_ref, *, inv_m):
    scale, shift = _bn_coeffs(s_ref, ss_ref, g_ref, b_ref, inv_m)
    y = y2_ref[0] * scale + shift + res_ref[0]
    o_ref[0] = jnp.maximum(y, 0.0).astype(o_ref.dtype)


def kernel(x, w1, cb1, g1, b1, w2, cb2, g2, b2):
    del cb1, cb2
    n, c, h, w = x.shape
    wp = w + 2
    m = h * wp
    rows_img = (h + 2) * wp
    need = rows_img + 2
    rows_pad = -(-need // 8) * 8

    x_nhwc = jnp.transpose(x, (0, 2, 3, 1)).astype(jnp.float32)
    xp = jnp.pad(x_nhwc, ((0, 0), (1, 1), (1, 1), (0, 0)))
    xp = xp.reshape(n, rows_img, c)
    xp = jnp.pad(xp, ((0, 0), (0, rows_pad - rows_img), (0, 0)))
    xp = xp.astype(jnp.bfloat16)
    res = jnp.pad(x_nhwc, ((0, 0), (0, 0), (0, 2), (0, 0)))
    res = res.reshape(n, m, c).astype(jnp.bfloat16)
    w1m = w1.reshape(9 * c, c).astype(jnp.bfloat16)
    w2m = w2.reshape(9 * c, c).astype(jnp.bfloat16)
    g1v = g1.astype(jnp.float32).reshape(1, c)
    b1v = b1.astype(jnp.float32).reshape(1, c)
    g2v = g2.astype(jnp.float32).reshape(1, c)
    b2v = b2.astype(jnp.float32).reshape(1, c)
    inv_m = 1.0 / (n * h * w)

    img_spec = pl.BlockSpec((1, rows_pad, c), lambda i: (i, 0, 0))
    row_spec = pl.BlockSpec((1, m, c), lambda i: (i, 0, 0))
    stat_spec = pl.BlockSpec((1, 1, c), lambda i: (i, 0, 0))
    stat_full = pl.BlockSpec((n, 1, c), lambda i: (0, 0, 0))
    w_spec = pl.BlockSpec((9 * c, c), lambda i: (0, 0))
    vec_spec = pl.BlockSpec((1, c), lambda i: (0, 0))
    params = pltpu.CompilerParams(dimension_semantics=("parallel",))
    f32 = jnp.float32
    bf16 = jnp.bfloat16

    y1, s1, ss1 = pl.pallas_call(
        functools.partial(_stage1, h=h, w=w, wp=wp, c=c),
        grid=(n,),
        in_specs=[img_spec, w_spec],
        out_specs=(img_spec, stat_spec, stat_spec),
        out_shape=(jax.ShapeDtypeStruct((n, rows_pad, c), bf16),
                   jax.ShapeDtypeStruct((n, 1, c), f32),
                   jax.ShapeDtypeStruct((n, 1, c), f32)),
        compiler_params=params,
    )(xp, w1m)

    y2, s2, ss2 = pl.pallas_call(
        functools.partial(_stage2, h=h, w=w, wp=wp, c=c, inv_m=inv_m),
        grid=(n,),
        in_specs=[img_spec, w_spec, stat_full, stat_full, vec_spec, vec_spec],
        out_specs=(row_spec, stat_spec, stat_spec),
        out_shape=(jax.ShapeDtypeStruct((n, m, c), bf16),
                   jax.ShapeDtypeStruct((n, 1, c), f32),
                   jax.ShapeDtypeStruct((n, 1, c), f32)),
        compiler_params=params,
    )(y1, w2m, s1, ss1, g1v, b1v)

    out = pl.pallas_call(
        functools.partial(_stage3, inv_m=inv_m),
        grid=(n,),
        in_specs=[row_spec, stat_full, stat_full, vec_spec, vec_spec, row_spec],
        out_specs=row_spec,
        out_shape=jax.ShapeDtypeStruct((n, m, c), f32),
        compiler_params=params,
    )(y2, s2, ss2, g2v, b2v, res)

    out = out.reshape(n, h, wp, c)[:, :, :w, :]
    return jnp.transpose(out, (0, 3, 1, 2))
